# baseline (speedup 1.0000x reference)
import jax
import jax.numpy as jnp
from jax import lax
from jax.experimental import pallas as pl
from jax.experimental.pallas import tpu as pltpu
from jax.experimental.pallas import tpu_sc as plsc

B = 16384
D_UA, D_GEO, D_ME = 16, 8, 4
D_OUT = D_UA + D_GEO + D_ME
N_UA, N_GEO, N_ME = 100, 50, 5
NC, NS = 2, 16
NW = NC * NS
BPW = B // NW
BLK = 16
NBLK = BPW // BLK
OUT_W = BPW * D_OUT
HALF_W = OUT_W // 2


def _emb_body(ua_id, geo_id, me_id, ua_t, geo_t, me_t, out,
              ua_i_v, geo_i_v, me_i_v, ua_tv, geo_tv, me_tv, out_v,
              s0, s1, s2):
    wid = lax.axis_index("s") * NC + lax.axis_index("c")
    base = wid * BPW

    c0 = pltpu.async_copy(ua_id.at[pl.ds(base, BPW)], ua_i_v, s0)
    c1 = pltpu.async_copy(geo_id.at[pl.ds(base, BPW)], geo_i_v, s1)
    c2 = pltpu.async_copy(me_id.at[pl.ds(base, BPW)], me_i_v, s2)
    t0 = pltpu.async_copy(ua_t, ua_tv, s0)
    t1 = pltpu.async_copy(geo_t, geo_tv, s1)
    t2 = pltpu.async_copy(me_t, me_tv, s2)
    c0.wait()
    c1.wait()
    c2.wait()
    t0.wait()
    t1.wait()
    t2.wait()

    iota = lax.iota(jnp.int32, 16)
    iota28 = iota * D_OUT

    def blk_body(b, carry):
        off = b * BLK
        ids_ua = ua_i_v[pl.ds(off, BLK)]
        ids_geo = geo_i_v[pl.ds(off, BLK)]
        ids_me = me_i_v[pl.ds(off, BLK)]
        rowbase = off * D_OUT + iota28
        for c in range(D_UA):
            vals = plsc.load_gather(ua_tv, [ids_ua + c * N_UA])
            plsc.store_scatter(out_v, [rowbase + c], vals)
        for c in range(D_GEO):
            vals = plsc.load_gather(geo_tv, [ids_geo + c * N_GEO])
            plsc.store_scatter(out_v, [rowbase + (D_UA + c)], vals)
        for c in range(D_ME):
            vals = plsc.load_gather(me_tv, [ids_me + c * N_ME])
            plsc.store_scatter(out_v, [rowbase + (D_UA + D_GEO + c)], vals)
        return carry

    lax.fori_loop(0, NBLK // 2, blk_body, 0, unroll=2)
    w0 = pltpu.async_copy(out_v.at[pl.ds(0, HALF_W)],
                          out.at[pl.ds(base * D_OUT, HALF_W)], s0)
    lax.fori_loop(NBLK // 2, NBLK, blk_body, 0, unroll=2)
    w1 = pltpu.async_copy(out_v.at[pl.ds(HALF_W, HALF_W)],
                          out.at[pl.ds(base * D_OUT + HALF_W, HALF_W)], s1)
    w0.wait()
    w1.wait()


_mesh = plsc.VectorSubcoreMesh(core_axis_name="c", subcore_axis_name="s")

_emb_call = pl.kernel(
    _emb_body,
    out_type=jax.ShapeDtypeStruct((B * D_OUT,), jnp.float32),
    mesh=_mesh,
    scratch_types=[
        pltpu.VMEM((BPW,), jnp.int32),
        pltpu.VMEM((BPW,), jnp.int32),
        pltpu.VMEM((BPW,), jnp.int32),
        pltpu.VMEM((N_UA * D_UA,), jnp.float32),
        pltpu.VMEM((N_GEO * D_GEO,), jnp.float32),
        pltpu.VMEM((N_ME * D_ME,), jnp.float32),
        pltpu.VMEM((OUT_W,), jnp.float32),
        pltpu.SemaphoreType.DMA,
        pltpu.SemaphoreType.DMA,
        pltpu.SemaphoreType.DMA,
    ],
    compiler_params=pltpu.CompilerParams(needs_layout_passes=False),
)


@jax.jit
def kernel(ua_id, geo_id, method_id, ua_table, geo_table, method_table):
    flat = _emb_call(
        ua_id.astype(jnp.int32),
        geo_id.astype(jnp.int32),
        method_id.astype(jnp.int32),
        ua_table.T.reshape(-1), geo_table.T.reshape(-1),
        method_table.T.reshape(-1),
    )
    return flat.reshape(B, D_OUT)
